# baseline (speedup 1.0000x reference)
import functools
import math

import jax
import jax.numpy as jnp
from jax import lax
from jax.experimental import pallas as pl
from jax.experimental.pallas import tpu as pltpu
from jax.experimental.pallas import tpu_sc as plsc

D_MODEL = 256
VOCAB = 5
SCALE = math.sqrt(D_MODEL)

NC = 2
NS = 16
NW = NC * NS
LANES = 16
PAIRS_PER_STEP = 16

HEAD_FRAC_NUM = 1
HEAD_FRAC_DEN = 2
TC_BLK = 1024


def _make_sc_kernel(B, R):
    n_pairs = (B - R) // 2
    p_per_w = n_pairs // NW
    n_steps = p_per_w // PAIRS_PER_STEP
    pair0 = R // 2
    mesh = plsc.VectorSubcoreMesh(core_axis_name="c", subcore_axis_name="s")

    @functools.partial(
        pl.kernel,
        mesh=mesh,
        out_type=jax.ShapeDtypeStruct((B, D_MODEL), jnp.float32),
        scratch_types=[
            pltpu.VMEM((VOCAB, D_MODEL), jnp.float32),
            pltpu.VMEM((VOCAB * VOCAB, 2, D_MODEL), jnp.float32),
            pltpu.VMEM((2 * p_per_w,), jnp.int32),
            pltpu.SemaphoreType.DMA,
        ],
    )
    def emb(table_hbm, idx_hbm, out_hbm, table_v, pt, idx_v, wsem):
        cid = lax.axis_index("c")
        sid = lax.axis_index("s")
        wid = sid * NC + cid
        base = pair0 + wid * p_per_w

        pltpu.sync_copy(table_hbm, table_v)
        for r in range(VOCAB):
            for j in range(D_MODEL // LANES):
                sl = pl.ds(j * LANES, LANES)
                table_v[r, sl] = table_v[r, sl] * SCALE
        for a in range(VOCAB):
            for b in range(VOCAB):
                for j in range(D_MODEL // LANES):
                    sl = pl.ds(j * LANES, LANES)
                    pt[a * VOCAB + b, 0, sl] = table_v[a, sl]
                    pt[a * VOCAB + b, 1, sl] = table_v[b, sl]

        pltpu.sync_copy(idx_hbm.at[pl.ds(2 * base, 2 * p_per_w)], idx_v)

        def take16(v, idx16):
            dnums = lax.GatherDimensionNumbers(
                offset_dims=(), collapsed_slice_dims=(0,),
                start_index_map=(0,))
            return lax.gather(
                v, idx16[:, None], dnums, slice_sizes=(1,),
                mode=lax.GatherScatterMode.PROMISE_IN_BOUNDS)

        lane = lax.iota(jnp.int32, LANES)
        sel_a = (lane * 2) & (LANES - 1)
        sel_b = (lane * 2 + 1) & (LANES - 1)
        lo = lane < (LANES // 2)

        @plsc.parallel_loop(0, n_steps, unroll=2)
        def step(i):
            i0 = i * PAIRS_PER_STEP
            v0 = idx_v[pl.ds(2 * i0, LANES)]
            v1 = idx_v[pl.ds(2 * i0 + LANES, LANES)]
            a16 = jnp.where(lo, take16(v0, sel_a), take16(v1, sel_a))
            b16 = jnp.where(lo, take16(v0, sel_b), take16(v1, sel_b))
            pid16 = a16 * VOCAB + b16
            for k in range(PAIRS_PER_STEP):
                r = pid16[k]
                pltpu.async_copy(
                    pt.at[r],
                    out_hbm.at[pl.ds(2 * (base + i0 + k), 2)],
                    wsem,
                )

        def drain(i, _):
            for k in range(PAIRS_PER_STEP):
                pltpu.make_async_copy(
                    pt.at[0],
                    out_hbm.at[pl.ds(2 * base, 2)],
                    wsem,
                ).wait()
            return _

        lax.fori_loop(0, n_steps, drain, 0, unroll=False)

    return emb


def _make_tc_kernel(B, R):
    def body(idx_ref, tbl_ref, _, out_ref):
        ids = idx_ref[...]
        oh = (ids[:, None]
              == lax.broadcasted_iota(jnp.int32, (1, VOCAB), 1))
        out_ref[...] = jnp.dot(
            oh.astype(jnp.float32), tbl_ref[...],
            preferred_element_type=jnp.float32) * SCALE

    return pl.pallas_call(
        body,
        grid=(R // TC_BLK,),
        in_specs=[
            pl.BlockSpec((TC_BLK,), lambda i: (i,)),
            pl.BlockSpec((VOCAB, D_MODEL), lambda i: (0, 0)),
            pl.BlockSpec(memory_space=pl.ANY),
        ],
        out_specs=pl.BlockSpec((TC_BLK, D_MODEL), lambda i: (i, 0)),
        out_shape=jax.ShapeDtypeStruct((B, D_MODEL), jnp.float32),
        input_output_aliases={2: 0},
    )


def kernel(x, table):
    B0, B1 = x.shape
    B = B0 * B1
    align = 2 * NW * PAIRS_PER_STEP
    R = (B * HEAD_FRAC_NUM // HEAD_FRAC_DEN) // align * align
    idx = x.reshape(B).astype(jnp.int32)
    out_sc = _make_sc_kernel(B, R)(table, idx)
    out = _make_tc_kernel(B, R)(idx, table, out_sc)
    return out.reshape(B0, B1, D_MODEL)


# speedup vs baseline: 1.2713x; 1.2713x over previous
import functools
import math

import jax
import jax.numpy as jnp
from jax import lax
from jax.experimental import pallas as pl
from jax.experimental.pallas import tpu as pltpu
from jax.experimental.pallas import tpu_sc as plsc

D_MODEL = 256
VOCAB = 5
SCALE = math.sqrt(D_MODEL)

NC = 2
NS = 16
NW = NC * NS
LANES = 16
PAIRS_PER_STEP = 16


def _make_kernel(B):
    n_pairs = B // 2
    p_per_w = n_pairs // NW
    n_steps = p_per_w // PAIRS_PER_STEP
    mesh = plsc.VectorSubcoreMesh(core_axis_name="c", subcore_axis_name="s")

    @functools.partial(
        pl.kernel,
        mesh=mesh,
        out_type=jax.ShapeDtypeStruct((B, D_MODEL), jnp.float32),
        scratch_types=[
            pltpu.VMEM((VOCAB, D_MODEL), jnp.float32),
            pltpu.VMEM((VOCAB * VOCAB, 2, D_MODEL), jnp.float32),
            pltpu.VMEM((2 * p_per_w,), jnp.int32),
            pltpu.SemaphoreType.DMA,
        ],
    )
    def emb(table_hbm, idx_hbm, out_hbm, table_v, pt, idx_v, wsem):
        cid = lax.axis_index("c")
        sid = lax.axis_index("s")
        wid = sid * NC + cid
        base = wid * p_per_w

        idx_cp = pltpu.async_copy(
            idx_hbm.at[pl.ds(2 * base, 2 * p_per_w)], idx_v, wsem)

        pltpu.sync_copy(table_hbm, table_v)
        for r in range(VOCAB):
            for j in range(D_MODEL // LANES):
                sl = pl.ds(j * LANES, LANES)
                table_v[r, sl] = table_v[r, sl] * SCALE
        for a in range(VOCAB):
            for b in range(VOCAB):
                for j in range(D_MODEL // LANES):
                    sl = pl.ds(j * LANES, LANES)
                    pt[a * VOCAB + b, 0, sl] = table_v[a, sl]
                    pt[a * VOCAB + b, 1, sl] = table_v[b, sl]

        idx_cp.wait()

        def take16(v, idx16):
            dnums = lax.GatherDimensionNumbers(
                offset_dims=(), collapsed_slice_dims=(0,),
                start_index_map=(0,))
            return lax.gather(
                v, idx16[:, None], dnums, slice_sizes=(1,),
                mode=lax.GatherScatterMode.PROMISE_IN_BOUNDS)

        lane = lax.iota(jnp.int32, LANES)
        sel_a = (lane * 2) & (LANES - 1)
        sel_b = (lane * 2 + 1) & (LANES - 1)
        lo = lane < (LANES // 2)

        def step(i, _):
            i0 = i * PAIRS_PER_STEP
            v0 = idx_v[pl.ds(2 * i0, LANES)]
            v1 = idx_v[pl.ds(2 * i0 + LANES, LANES)]
            a16 = jnp.where(
                lo,
                take16(v0, sel_a),
                take16(v1, sel_a),
            )
            b16 = jnp.where(
                lo,
                take16(v0, sel_b),
                take16(v1, sel_b),
            )
            pid16 = a16 * VOCAB + b16
            for k in range(PAIRS_PER_STEP):
                r = pid16[k]
                pltpu.async_copy(
                    pt.at[r],
                    out_hbm.at[pl.ds(2 * (base + i0 + k), 2)],
                    wsem,
                )
            return _

        lax.fori_loop(0, n_steps, step, 0, unroll=False)


        def drain(i, _):
            for k in range(PAIRS_PER_STEP):
                pltpu.make_async_copy(
                    pt.at[0],
                    out_hbm.at[pl.ds(2 * base, 2)],
                    wsem,
                ).wait()
            return _

        lax.fori_loop(0, n_steps, drain, 0, unroll=False)

    return emb


def kernel(x, table):
    B0, B1 = x.shape
    B = B0 * B1
    idx = x.reshape(B).astype(jnp.int32)
    out = _make_kernel(B)(table, idx)
    return out.reshape(B0, B1, D_MODEL)
